# baseline (speedup 1.0000x reference)
import jax
import jax.numpy as jnp
from jax import lax
from jax.experimental import pallas as pl

_E = 8
_BT = 4096


def _top2(logits):
    lt = logits.T
    sub = lax.broadcasted_iota(jnp.int32, lt.shape, 0)
    m1 = jnp.max(lt, axis=0, keepdims=True)
    i1 = jnp.min(jnp.where(lt == m1, sub, _E), axis=0, keepdims=True)
    masked = jnp.where(sub == i1, -jnp.inf, lt)
    m2 = jnp.max(masked, axis=0, keepdims=True)
    i2 = jnp.min(jnp.where(masked == m2, sub, _E), axis=0, keepdims=True)
    e = jnp.exp(m2 - m1)
    den = 1.0 + e
    return (jnp.concatenate([1.0 / den, e / den], axis=0).T,
            jnp.concatenate([i1, i2], axis=0).T)


def _router_block(x_ref, w_ref, logits_ref, prob_ref, idx_ref):
    logits = lax.dot_general(
        x_ref[...], w_ref[...], (((1,), (1,)), ((), ())),
        preferred_element_type=jnp.float32,
    )
    logits_ref[...] = logits
    prob, idx = _top2(logits)
    prob_ref[...] = prob
    idx_ref[...] = idx


def kernel(hidden_states, weight):
    n_tokens, hidden = hidden_states.shape
    n_experts = weight.shape[0]
    grid = (n_tokens // _BT,)
    return pl.pallas_call(
        _router_block,
        grid=grid,
        in_specs=[
            pl.BlockSpec((_BT, hidden), lambda i: (i, 0)),
            pl.BlockSpec((n_experts, hidden), lambda i: (0, 0)),
        ],
        out_specs=[
            pl.BlockSpec((_BT, n_experts), lambda i: (i, 0)),
            pl.BlockSpec((_BT, 2), lambda i: (i, 0)),
            pl.BlockSpec((_BT, 2), lambda i: (i, 0)),
        ],
        out_shape=[
            jax.ShapeDtypeStruct((n_tokens, n_experts), jnp.float32),
            jax.ShapeDtypeStruct((n_tokens, 2), jnp.float32),
            jax.ShapeDtypeStruct((n_tokens, 2), jnp.int32),
        ],
    )(hidden_states, weight)


# speedup vs baseline: 1.6333x; 1.6333x over previous
import jax
import jax.numpy as jnp
from jax import lax
from jax.experimental import pallas as pl

_E = 8
_BT = 4096


def _top2(logits):
    lt = logits.T
    sub = lax.broadcasted_iota(jnp.int32, lt.shape, 0)
    m1 = jnp.max(lt, axis=0, keepdims=True)
    i1 = jnp.min(jnp.where(lt == m1, sub, _E), axis=0, keepdims=True)
    masked = jnp.where(sub == i1, -jnp.inf, lt)
    m2 = jnp.max(masked, axis=0, keepdims=True)
    i2 = jnp.min(jnp.where(masked == m2, sub, _E), axis=0, keepdims=True)
    e = jnp.exp(m2 - m1)
    den = 1.0 + e
    return (jnp.concatenate([1.0 / den, e / den], axis=0),
            jnp.concatenate([i1, i2], axis=0))


def _router_block(x_ref, w_ref, logits_ref, prob_ref, idx_ref):
    logits = lax.dot_general(
        x_ref[...], w_ref[...], (((1,), (1,)), ((), ())),
        preferred_element_type=jnp.float32,
    )
    logits_ref[...] = logits
    prob, idx = _top2(logits)
    prob_ref[...] = prob
    idx_ref[...] = idx


def kernel(hidden_states, weight):
    n_tokens, hidden = hidden_states.shape
    n_experts = weight.shape[0]
    grid = (n_tokens // _BT,)
    logits, prob_t, idx_t = pl.pallas_call(
        _router_block,
        grid=grid,
        in_specs=[
            pl.BlockSpec((_BT, hidden), lambda i: (i, 0)),
            pl.BlockSpec((n_experts, hidden), lambda i: (0, 0)),
        ],
        out_specs=[
            pl.BlockSpec((_BT, n_experts), lambda i: (i, 0)),
            pl.BlockSpec((2, _BT), lambda i: (0, i)),
            pl.BlockSpec((2, _BT), lambda i: (0, i)),
        ],
        out_shape=[
            jax.ShapeDtypeStruct((n_tokens, n_experts), jnp.float32),
            jax.ShapeDtypeStruct((2, n_tokens), jnp.float32),
            jax.ShapeDtypeStruct((2, n_tokens), jnp.int32),
        ],
    )(hidden_states, weight)
    return (logits, prob_t.T, idx_t.T)
